# baseline (speedup 1.0000x reference)
import functools

import jax
import jax.numpy as jnp
from jax import lax
from jax.experimental import pallas as pl
from jax.experimental.pallas import tpu as pltpu
from jax.experimental.pallas import tpu_sc as plsc

N = 8388608
NC = 2
NS = 16
L = 16
NW = NC * NS
PER_W = N // NW
CHUNK = 16384
NCHUNK = PER_W // CHUNK
SLICES = CHUNK // L

_mesh = plsc.VectorSubcoreMesh(core_axis_name="c", subcore_axis_name="s")


@functools.partial(
    pl.kernel,
    out_type=jax.ShapeDtypeStruct((NW, 4, L), jnp.float32),
    mesh=_mesh,
    scratch_types=[
        pltpu.VMEM((2, CHUNK), jnp.float32),
        pltpu.VMEM((2, CHUNK), jnp.int32),
        pltpu.VMEM((2, CHUNK), jnp.int32),
        pltpu.VMEM((4, L), jnp.float32),
        pltpu.SemaphoreType.DMA,
        pltpu.SemaphoreType.DMA,
    ],
)
def _partial_sums(yp_hbm, s_hbm, g_hbm, out_hbm,
                  yp_buf, s_buf, g_buf, res_v, sem0, sem1):
    wid = lax.axis_index("s") * NC + lax.axis_index("c")
    base = wid * PER_W
    sems = (sem0, sem1)

    def start(c, slot):
        off = base + c * CHUNK
        return (
            pltpu.async_copy(yp_hbm.at[pl.ds(off, CHUNK)], yp_buf.at[slot],
                             sems[slot]),
            pltpu.async_copy(s_hbm.at[pl.ds(off, CHUNK)], s_buf.at[slot],
                             sems[slot]),
            pltpu.async_copy(g_hbm.at[pl.ds(off, CHUNK)], g_buf.at[slot],
                             sems[slot]),
        )

    def chunk_body(slot, accs):
        def body(i, accs):
            acc_sv, acc_cv, acc_s1, acc_c1 = accs
            o = i * L
            yp = yp_buf[slot, pl.ds(o, L)]
            sv = s_buf[slot, pl.ds(o, L)]
            gv = g_buf[slot, pl.ds(o, L)]
            m1 = gv & sv
            gf = gv.astype(jnp.float32)
            m1f = m1.astype(jnp.float32)
            return (acc_sv + yp * gf, acc_cv + gf,
                    acc_s1 + yp * m1f, acc_c1 + m1f)
        return lax.fori_loop(0, SLICES, body, accs)

    zero = jnp.zeros((L,), jnp.float32)
    accs = (zero, zero, zero, zero)
    inflight = [None, None]
    inflight[0] = start(0, 0)
    for c in range(NCHUNK):
        if c + 1 < NCHUNK:
            inflight[(c + 1) % 2] = start(c + 1, (c + 1) % 2)
        for cp in inflight[c % 2]:
            cp.wait()
        accs = chunk_body(c % 2, accs)

    acc_sv, acc_cv, acc_s1, acc_c1 = accs
    res_v[0, :] = acc_sv - acc_s1
    res_v[1, :] = acc_cv - acc_c1
    res_v[2, :] = acc_s1
    res_v[3, :] = acc_c1
    pltpu.sync_copy(res_v, out_hbm.at[wid])


def kernel(y_pred, s, y_gt):
    y_pred = y_pred.reshape(-1)
    s = s.reshape(-1).astype(jnp.int32)
    y_gt = y_gt.reshape(-1).astype(jnp.int32)
    p = _partial_sums(y_pred, s, y_gt)
    t = jnp.sum(p, axis=(0, 2))
    return jnp.abs(t[0] / t[1] - t[2] / t[3])
